# baseline (speedup 1.0000x reference)
import jax
import jax.numpy as jnp
from jax import lax
from jax.experimental import pallas as pl
from jax.experimental.pallas import tpu as pltpu
from jax.experimental.pallas import tpu_sc as plsc

ORG_VOCAB = 100000
EXTRA_VOCAB = 256
FULL_VOCAB = ORG_VOCAB + EXTRA_VOCAB
EMBED_DIM = 1024
MAX_L = 8
RANK = 16
T = 16384

NC = 2
NS = 16
NW = NC * NS
B_PER_W = T // NW
GCHUNK = 128
WCH = 32
NCH = B_PER_W // WCH
NBUF = 3

TOK_BLK = 256
NBLK = T // TOK_BLK


def _sc_body(x_hbm, l_hbm, lora_a_hbm, weight_hbm,
             arows_hbm, base_hbm,
             x_v, l_v, idx_v, bidx_v, arows_v, buf0, buf1, buf2,
             asem, gs0, gs1, gs2, os0, os1, os2):
    wid = lax.axis_index("s") * NC + lax.axis_index("c")
    base = wid * B_PER_W
    pltpu.sync_copy(x_hbm.at[pl.ds(base, B_PER_W)], x_v)
    pltpu.sync_copy(l_hbm.at[pl.ds(base, B_PER_W)], l_v)

    def step(i, carry):
        xs = x_v[pl.ds(i * 16, 16)]
        ls = l_v[pl.ds(i * 16, 16)]
        idx_v[pl.ds(i * 16, 16)] = xs + ls * FULL_VOCAB
        extra = jnp.where(xs > ORG_VOCAB - 1, ls * EXTRA_VOCAB, 0)
        bidx_v[pl.ds(i * 16, 16)] = xs + extra
        return carry

    lax.fori_loop(0, B_PER_W // 16, step, 0)

    a_copies = []
    for j in range(B_PER_W // GCHUNK):
        a_copies.append(pltpu.async_copy(
            lora_a_hbm.at[idx_v.at[pl.ds(j * GCHUNK, GCHUNK)]],
            arows_v.at[pl.ds(j * GCHUNK, GCHUNK)],
            asem,
        ))

    bufs = [buf0, buf1, buf2]
    gsems = [gs0, gs1, gs2]
    osems = [os0, os1, os2]
    gcp = {}
    ocp = {}
    for c in range(NCH):
        b = c % NBUF
        if c >= NBUF:
            ocp[c - NBUF].wait()
        gcp[c] = pltpu.async_copy(
            weight_hbm.at[bidx_v.at[pl.ds(c * WCH, WCH)]], bufs[b], gsems[b])
        if c >= 1:
            pb = (c - 1) % NBUF
            gcp[c - 1].wait()
            ocp[c - 1] = pltpu.async_copy(
                bufs[pb], base_hbm.at[pl.ds(base + (c - 1) * WCH, WCH)],
                osems[pb])
    gcp[NCH - 1].wait()
    ocp[NCH - 1] = pltpu.async_copy(
        bufs[(NCH - 1) % NBUF],
        base_hbm.at[pl.ds(base + (NCH - 1) * WCH, WCH)],
        osems[(NCH - 1) % NBUF])
    for c in range(NCH - NBUF, NCH):
        ocp[c].wait()

    for cp in a_copies:
        cp.wait()
    pltpu.sync_copy(arows_v, arows_hbm.at[pl.ds(base, B_PER_W)])


def _sc_gather(x, lora_indices, lora_a_stacked_2d, weight):
    mesh = plsc.VectorSubcoreMesh(core_axis_name="c", subcore_axis_name="s")
    return pl.kernel(
        _sc_body,
        out_type=[
            jax.ShapeDtypeStruct((T, RANK), jnp.float32),
            jax.ShapeDtypeStruct((T, EMBED_DIM), jnp.float32),
        ],
        mesh=mesh,
        scratch_types=[
            pltpu.VMEM((B_PER_W,), jnp.int32),
            pltpu.VMEM((B_PER_W,), jnp.int32),
            pltpu.VMEM((B_PER_W,), jnp.int32),
            pltpu.VMEM((B_PER_W,), jnp.int32),
            pltpu.VMEM((B_PER_W, RANK), jnp.float32),
            pltpu.VMEM((WCH, EMBED_DIM), jnp.float32),
            pltpu.VMEM((WCH, EMBED_DIM), jnp.float32),
            pltpu.VMEM((WCH, EMBED_DIM), jnp.float32),
            pltpu.SemaphoreType.DMA,
            pltpu.SemaphoreType.DMA,
            pltpu.SemaphoreType.DMA,
            pltpu.SemaphoreType.DMA,
            pltpu.SemaphoreType.DMA,
            pltpu.SemaphoreType.DMA,
            pltpu.SemaphoreType.DMA,
        ],
        compiler_params=pltpu.CompilerParams(use_tc_tiling_on_sc=False),
    )(x, lora_indices, lora_a_stacked_2d, weight)


def _tc_body(a_ref, l_ref, bflat_ref, base_ref, out_ref):
    a = a_ref[...]
    at8 = jnp.concatenate([a] * MAX_L, axis=1)
    lv = l_ref[...]
    col = lax.broadcasted_iota(jnp.int32, (TOK_BLK, MAX_L * RANK), 1) // RANK
    a_scat = jnp.where(col == lv, at8, 0.0)
    lora = jnp.dot(a_scat, bflat_ref[...], preferred_element_type=jnp.float32)
    out_ref[...] = base_ref[...] + lora


def _tc_call(a_rows, l2d, bflat, base_out):
    return pl.pallas_call(
        _tc_body,
        grid=(NBLK,),
        in_specs=[
            pl.BlockSpec((TOK_BLK, RANK), lambda i: (i, 0)),
            pl.BlockSpec((TOK_BLK, 1), lambda i: (i, 0)),
            pl.BlockSpec((MAX_L * RANK, EMBED_DIM), lambda i: (0, 0)),
            pl.BlockSpec((TOK_BLK, EMBED_DIM), lambda i: (i, 0)),
        ],
        out_specs=pl.BlockSpec((TOK_BLK, EMBED_DIM), lambda i: (i, 0)),
        out_shape=jax.ShapeDtypeStruct((T, EMBED_DIM), jnp.float32),
    )(a_rows, l2d, bflat, base_out)


def kernel(x, lora_indices, weight, lora_a_stacked_2d, lora_b_stacked):
    a_rows, base_out = _sc_gather(x, lora_indices, lora_a_stacked_2d, weight)
    bflat = jnp.transpose(lora_b_stacked[:, 0], (0, 2, 1)).reshape(
        MAX_L * RANK, EMBED_DIM)
    l2d = lora_indices.reshape(T, 1)
    return _tc_call(a_rows, l2d, bflat, base_out)


# speedup vs baseline: 1.5304x; 1.5304x over previous
import jax
import jax.numpy as jnp
from jax import lax
from jax.experimental import pallas as pl
from jax.experimental.pallas import tpu as pltpu
from jax.experimental.pallas import tpu_sc as plsc

ORG_VOCAB = 100000
EXTRA_VOCAB = 256
FULL_VOCAB = ORG_VOCAB + EXTRA_VOCAB
EMBED_DIM = 1024
MAX_L = 8
RANK = 16
T = 16384

NC = 2
NS = 16
NW = NC * NS
B_PER_W = T // NW
GCHUNK = 128
WCH = 32
NCH = B_PER_W // WCH
NBUF = 3

TOK_BLK = 256
NBLK = T // TOK_BLK


def _sc_a_body(x_hbm, l_hbm, lora_a_hbm, arows_hbm, bidx_hbm,
               x_v, l_v, idx_v, bidx_v, arows_v, sem):
    wid = lax.axis_index("s") * NC + lax.axis_index("c")
    base = wid * B_PER_W
    pltpu.sync_copy(x_hbm.at[pl.ds(base, B_PER_W)], x_v)
    pltpu.sync_copy(l_hbm.at[pl.ds(base, B_PER_W)], l_v)

    def step(i, carry):
        xs = x_v[pl.ds(i * 16, 16)]
        ls = l_v[pl.ds(i * 16, 16)]
        idx_v[pl.ds(i * 16, 16)] = xs + ls * FULL_VOCAB
        extra = jnp.where(xs > ORG_VOCAB - 1, ls * EXTRA_VOCAB, 0)
        bidx_v[pl.ds(i * 16, 16)] = xs + extra
        return carry

    lax.fori_loop(0, B_PER_W // 16, step, 0)

    copies = []
    for j in range(B_PER_W // GCHUNK):
        copies.append(pltpu.async_copy(
            lora_a_hbm.at[idx_v.at[pl.ds(j * GCHUNK, GCHUNK)]],
            arows_v.at[pl.ds(j * GCHUNK, GCHUNK)],
            sem,
        ))
    for cp in copies:
        cp.wait()

    pltpu.sync_copy(arows_v, arows_hbm.at[pl.ds(base, B_PER_W)])
    pltpu.sync_copy(bidx_v, bidx_hbm.at[pl.ds(base, B_PER_W)])


def _sc_a_gather(x, lora_indices, lora_a_stacked_2d):
    mesh = plsc.VectorSubcoreMesh(core_axis_name="c", subcore_axis_name="s")
    return pl.kernel(
        _sc_a_body,
        out_type=[
            jax.ShapeDtypeStruct((T, RANK), jnp.float32),
            jax.ShapeDtypeStruct((T,), jnp.int32),
        ],
        mesh=mesh,
        scratch_types=[
            pltpu.VMEM((B_PER_W,), jnp.int32),
            pltpu.VMEM((B_PER_W,), jnp.int32),
            pltpu.VMEM((B_PER_W,), jnp.int32),
            pltpu.VMEM((B_PER_W,), jnp.int32),
            pltpu.VMEM((B_PER_W, RANK), jnp.float32),
            pltpu.SemaphoreType.DMA,
        ],
        compiler_params=pltpu.CompilerParams(use_tc_tiling_on_sc=False),
    )(x, lora_indices, lora_a_stacked_2d)


def _sc_w_body(bidx_hbm, weight_hbm, base_hbm,
               bidx_v, buf0, buf1, buf2, gs0, gs1, gs2, os0, os1, os2):
    wid = lax.axis_index("s") * NC + lax.axis_index("c")
    base = wid * B_PER_W
    pltpu.sync_copy(bidx_hbm.at[pl.ds(base, B_PER_W)], bidx_v)

    bufs = [buf0, buf1, buf2]
    gsems = [gs0, gs1, gs2]
    osems = [os0, os1, os2]
    gcp = {}
    ocp = {}
    for c in range(NCH):
        b = c % NBUF
        if c >= NBUF:
            ocp[c - NBUF].wait()
        gcp[c] = pltpu.async_copy(
            weight_hbm.at[bidx_v.at[pl.ds(c * WCH, WCH)]], bufs[b], gsems[b])
        if c >= 1:
            pb = (c - 1) % NBUF
            gcp[c - 1].wait()
            ocp[c - 1] = pltpu.async_copy(
                bufs[pb], base_hbm.at[pl.ds(base + (c - 1) * WCH, WCH)],
                osems[pb])
    gcp[NCH - 1].wait()
    ocp[NCH - 1] = pltpu.async_copy(
        bufs[(NCH - 1) % NBUF],
        base_hbm.at[pl.ds(base + (NCH - 1) * WCH, WCH)],
        osems[(NCH - 1) % NBUF])
    for c in range(NCH - NBUF, NCH):
        ocp[c].wait()


def _sc_w_gather(bidx, weight):
    mesh = plsc.VectorSubcoreMesh(core_axis_name="c", subcore_axis_name="s")
    return pl.kernel(
        _sc_w_body,
        out_type=jax.ShapeDtypeStruct((T, EMBED_DIM), jnp.float32),
        mesh=mesh,
        scratch_types=[
            pltpu.VMEM((B_PER_W,), jnp.int32),
            pltpu.VMEM((WCH, EMBED_DIM), jnp.float32),
            pltpu.VMEM((WCH, EMBED_DIM), jnp.float32),
            pltpu.VMEM((WCH, EMBED_DIM), jnp.float32),
            pltpu.SemaphoreType.DMA,
            pltpu.SemaphoreType.DMA,
            pltpu.SemaphoreType.DMA,
            pltpu.SemaphoreType.DMA,
            pltpu.SemaphoreType.DMA,
            pltpu.SemaphoreType.DMA,
        ],
    )(bidx, weight)


def _tc_body(a_ref, l_ref, bflat_ref, base_ref, out_ref):
    a = a_ref[...]
    at8 = jnp.concatenate([a] * MAX_L, axis=1)
    lv = l_ref[...]
    col = lax.broadcasted_iota(jnp.int32, (TOK_BLK, MAX_L * RANK), 1) // RANK
    a_scat = jnp.where(col == lv, at8, 0.0)
    lora = jnp.dot(a_scat, bflat_ref[...], preferred_element_type=jnp.float32)
    out_ref[...] = base_ref[...] + lora


def _tc_call(a_rows, l2d, bflat, base_out):
    return pl.pallas_call(
        _tc_body,
        grid=(NBLK,),
        in_specs=[
            pl.BlockSpec((TOK_BLK, RANK), lambda i: (i, 0)),
            pl.BlockSpec((TOK_BLK, 1), lambda i: (i, 0)),
            pl.BlockSpec((MAX_L * RANK, EMBED_DIM), lambda i: (0, 0)),
            pl.BlockSpec((TOK_BLK, EMBED_DIM), lambda i: (i, 0)),
        ],
        out_specs=pl.BlockSpec((TOK_BLK, EMBED_DIM), lambda i: (i, 0)),
        out_shape=jax.ShapeDtypeStruct((T, EMBED_DIM), jnp.float32),
    )(a_rows, l2d, bflat, base_out)


def kernel(x, lora_indices, weight, lora_a_stacked_2d, lora_b_stacked):
    a_rows, bidx = _sc_a_gather(x, lora_indices, lora_a_stacked_2d)
    base_out = _sc_w_gather(bidx, weight)
    bflat = jnp.transpose(lora_b_stacked[:, 0], (0, 2, 1)).reshape(
        MAX_L * RANK, EMBED_DIM)
    l2d = lora_indices.reshape(T, 1)
    return _tc_call(a_rows, l2d, bflat, base_out)


# speedup vs baseline: 1.6289x; 1.0644x over previous
import jax
import jax.numpy as jnp
from jax import lax
from jax.experimental import pallas as pl
from jax.experimental.pallas import tpu as pltpu
from jax.experimental.pallas import tpu_sc as plsc

ORG_VOCAB = 100000
EXTRA_VOCAB = 256
FULL_VOCAB = ORG_VOCAB + EXTRA_VOCAB
EMBED_DIM = 1024
MAX_L = 8
RANK = 16
T = 16384

NC = 2
NS = 16
NW = NC * NS
B_PER_W = T // NW
GCHUNK = 128
WCH = 32
NCH = B_PER_W // WCH
NBUF = 3

TOK_BLK = 1024
NBLK = T // TOK_BLK


def _sc_a_body(x_hbm, l_hbm, lora_a_hbm, arows_hbm, bidx_hbm,
               x_v, l_v, idx_v, bidx_v, arows_v, sem):
    wid = lax.axis_index("s") * NC + lax.axis_index("c")
    base = wid * B_PER_W
    pltpu.sync_copy(x_hbm.at[pl.ds(base, B_PER_W)], x_v)
    pltpu.sync_copy(l_hbm.at[pl.ds(base, B_PER_W)], l_v)

    def step(i, carry):
        xs = x_v[pl.ds(i * 16, 16)]
        ls = l_v[pl.ds(i * 16, 16)]
        idx_v[pl.ds(i * 16, 16)] = xs + ls * FULL_VOCAB
        extra = jnp.where(xs > ORG_VOCAB - 1, ls * EXTRA_VOCAB, 0)
        bidx_v[pl.ds(i * 16, 16)] = xs + extra
        return carry

    lax.fori_loop(0, B_PER_W // 16, step, 0)

    copies = []
    for j in range(B_PER_W // GCHUNK):
        copies.append(pltpu.async_copy(
            lora_a_hbm.at[idx_v.at[pl.ds(j * GCHUNK, GCHUNK)]],
            arows_v.at[pl.ds(j * GCHUNK, GCHUNK)],
            sem,
        ))
    for cp in copies:
        cp.wait()

    pltpu.sync_copy(arows_v, arows_hbm.at[pl.ds(base, B_PER_W)])
    pltpu.sync_copy(bidx_v, bidx_hbm.at[pl.ds(base, B_PER_W)])


def _sc_a_gather(x, lora_indices, lora_a_stacked_2d):
    mesh = plsc.VectorSubcoreMesh(core_axis_name="c", subcore_axis_name="s")
    return pl.kernel(
        _sc_a_body,
        out_type=[
            jax.ShapeDtypeStruct((T, RANK), jnp.float32),
            jax.ShapeDtypeStruct((T,), jnp.int32),
        ],
        mesh=mesh,
        scratch_types=[
            pltpu.VMEM((B_PER_W,), jnp.int32),
            pltpu.VMEM((B_PER_W,), jnp.int32),
            pltpu.VMEM((B_PER_W,), jnp.int32),
            pltpu.VMEM((B_PER_W,), jnp.int32),
            pltpu.VMEM((B_PER_W, RANK), jnp.float32),
            pltpu.SemaphoreType.DMA,
        ],
        compiler_params=pltpu.CompilerParams(use_tc_tiling_on_sc=False),
    )(x, lora_indices, lora_a_stacked_2d)


def _sc_w_body(bidx_hbm, weight_hbm, base_hbm,
               bidx_v, buf0, buf1, buf2, gs0, gs1, gs2, os0, os1, os2):
    wid = lax.axis_index("s") * NC + lax.axis_index("c")
    base = wid * B_PER_W
    pltpu.sync_copy(bidx_hbm.at[pl.ds(base, B_PER_W)], bidx_v)

    bufs = [buf0, buf1, buf2]
    gsems = [gs0, gs1, gs2]
    osems = [os0, os1, os2]
    gcp = {}
    ocp = {}
    for c in range(NCH):
        b = c % NBUF
        if c >= NBUF:
            ocp[c - NBUF].wait()
        gcp[c] = pltpu.async_copy(
            weight_hbm.at[bidx_v.at[pl.ds(c * WCH, WCH)]], bufs[b], gsems[b])
        if c >= 1:
            pb = (c - 1) % NBUF
            gcp[c - 1].wait()
            ocp[c - 1] = pltpu.async_copy(
                bufs[pb], base_hbm.at[pl.ds(base + (c - 1) * WCH, WCH)],
                osems[pb])
    gcp[NCH - 1].wait()
    ocp[NCH - 1] = pltpu.async_copy(
        bufs[(NCH - 1) % NBUF],
        base_hbm.at[pl.ds(base + (NCH - 1) * WCH, WCH)],
        osems[(NCH - 1) % NBUF])
    for c in range(NCH - NBUF, NCH):
        ocp[c].wait()


def _sc_w_gather(bidx, weight):
    mesh = plsc.VectorSubcoreMesh(core_axis_name="c", subcore_axis_name="s")
    return pl.kernel(
        _sc_w_body,
        out_type=jax.ShapeDtypeStruct((T, EMBED_DIM), jnp.float32),
        mesh=mesh,
        scratch_types=[
            pltpu.VMEM((B_PER_W,), jnp.int32),
            pltpu.VMEM((WCH, EMBED_DIM), jnp.float32),
            pltpu.VMEM((WCH, EMBED_DIM), jnp.float32),
            pltpu.VMEM((WCH, EMBED_DIM), jnp.float32),
            pltpu.SemaphoreType.DMA,
            pltpu.SemaphoreType.DMA,
            pltpu.SemaphoreType.DMA,
            pltpu.SemaphoreType.DMA,
            pltpu.SemaphoreType.DMA,
            pltpu.SemaphoreType.DMA,
        ],
    )(bidx, weight)


def _tc_body(a_ref, l_ref, bflat_ref, base_ref, out_ref):
    a = a_ref[...]
    at8 = jnp.concatenate([a] * MAX_L, axis=1)
    lv = l_ref[...]
    col = lax.broadcasted_iota(jnp.int32, (TOK_BLK, MAX_L * RANK), 1) // RANK
    a_scat = jnp.where(col == lv, at8, 0.0)
    lora = jnp.dot(a_scat, bflat_ref[...], preferred_element_type=jnp.float32)
    out_ref[...] = base_ref[...] + lora


def _tc_call(a_rows, l2d, bflat, base_out):
    return pl.pallas_call(
        _tc_body,
        grid=(NBLK,),
        in_specs=[
            pl.BlockSpec((TOK_BLK, RANK), lambda i: (i, 0)),
            pl.BlockSpec((TOK_BLK, 1), lambda i: (i, 0)),
            pl.BlockSpec((MAX_L * RANK, EMBED_DIM), lambda i: (0, 0)),
            pl.BlockSpec((TOK_BLK, EMBED_DIM), lambda i: (i, 0)),
        ],
        out_specs=pl.BlockSpec((TOK_BLK, EMBED_DIM), lambda i: (i, 0)),
        out_shape=jax.ShapeDtypeStruct((T, EMBED_DIM), jnp.float32),
    )(a_rows, l2d, bflat, base_out)


def kernel(x, lora_indices, weight, lora_a_stacked_2d, lora_b_stacked):
    a_rows, bidx = _sc_a_gather(x, lora_indices, lora_a_stacked_2d)
    base_out = _sc_w_gather(bidx, weight)
    bflat = jnp.transpose(lora_b_stacked[:, 0], (0, 2, 1)).reshape(
        MAX_L * RANK, EMBED_DIM)
    l2d = lora_indices.reshape(T, 1)
    return _tc_call(a_rows, l2d, bflat, base_out)


# speedup vs baseline: 1.6410x; 1.0074x over previous
import jax
import jax.numpy as jnp
from jax import lax
from jax.experimental import pallas as pl
from jax.experimental.pallas import tpu as pltpu
from jax.experimental.pallas import tpu_sc as plsc

ORG_VOCAB = 100000
EXTRA_VOCAB = 256
FULL_VOCAB = ORG_VOCAB + EXTRA_VOCAB
EMBED_DIM = 1024
MAX_L = 8
RANK = 16
T = 16384

NC = 2
NS = 16
NW = NC * NS
B_PER_W = T // NW
GCHUNK = 128
WCH = 32
NCH = B_PER_W // WCH
NBUF = 3

TOK_BLK = 1024
NBLK = T // TOK_BLK


def _sc_a_body(x_hbm, l_hbm, lora_a_hbm, g_hbm, bidx_hbm,
               x_v, l_v, idx_v, bidx_v, arows_v, g_v, sem):
    wid = lax.axis_index("s") * NC + lax.axis_index("c")
    base = wid * B_PER_W
    pltpu.sync_copy(x_hbm.at[pl.ds(base, B_PER_W)], x_v)
    pltpu.sync_copy(l_hbm.at[pl.ds(base, B_PER_W)], l_v)

    def step(i, carry):
        xs = x_v[pl.ds(i * 16, 16)]
        ls = l_v[pl.ds(i * 16, 16)]
        idx_v[pl.ds(i * 16, 16)] = xs + ls * FULL_VOCAB
        extra = jnp.where(xs > ORG_VOCAB - 1, ls * EXTRA_VOCAB, 0)
        bidx_v[pl.ds(i * 16, 16)] = xs + extra
        return carry

    lax.fori_loop(0, B_PER_W // 16, step, 0)

    copies = []
    for j in range(B_PER_W // GCHUNK):
        copies.append(pltpu.async_copy(
            lora_a_hbm.at[idx_v.at[pl.ds(j * GCHUNK, GCHUNK)]],
            arows_v.at[pl.ds(j * GCHUNK, GCHUNK)],
            sem,
        ))
    for cp in copies:
        cp.wait()

    zeros16 = jnp.zeros((16,), jnp.float32)

    def gstep(gi, carry):
        lvec = l_v[pl.ds(gi * 16, 16)]
        for k in range(16):
            t = gi * 16 + k
            lt = lvec[k]
            arow = arows_v[t, pl.ds(0, RANK)]
            for j in range(MAX_L):
                g_v[t, pl.ds(j * RANK, RANK)] = zeros16
            g_v[t, pl.ds(lt * RANK, RANK)] = arow
        return carry

    lax.fori_loop(0, B_PER_W // 16, gstep, 0)

    pltpu.sync_copy(g_v, g_hbm.at[pl.ds(base, B_PER_W)])
    pltpu.sync_copy(bidx_v, bidx_hbm.at[pl.ds(base, B_PER_W)])


def _sc_a_gather(x, lora_indices, lora_a_stacked_2d):
    mesh = plsc.VectorSubcoreMesh(core_axis_name="c", subcore_axis_name="s")
    return pl.kernel(
        _sc_a_body,
        out_type=[
            jax.ShapeDtypeStruct((T, MAX_L * RANK), jnp.float32),
            jax.ShapeDtypeStruct((T,), jnp.int32),
        ],
        mesh=mesh,
        scratch_types=[
            pltpu.VMEM((B_PER_W,), jnp.int32),
            pltpu.VMEM((B_PER_W,), jnp.int32),
            pltpu.VMEM((B_PER_W,), jnp.int32),
            pltpu.VMEM((B_PER_W,), jnp.int32),
            pltpu.VMEM((B_PER_W, RANK), jnp.float32),
            pltpu.VMEM((B_PER_W, MAX_L * RANK), jnp.float32),
            pltpu.SemaphoreType.DMA,
        ],
        compiler_params=pltpu.CompilerParams(use_tc_tiling_on_sc=False),
    )(x, lora_indices, lora_a_stacked_2d)


def _sc_w_body(bidx_hbm, weight_hbm, base_hbm,
               bidx_v, buf0, buf1, buf2, gs0, gs1, gs2, os0, os1, os2):
    wid = lax.axis_index("s") * NC + lax.axis_index("c")
    base = wid * B_PER_W
    pltpu.sync_copy(bidx_hbm.at[pl.ds(base, B_PER_W)], bidx_v)

    bufs = [buf0, buf1, buf2]
    gsems = [gs0, gs1, gs2]
    osems = [os0, os1, os2]
    gcp = {}
    ocp = {}
    for c in range(NCH):
        b = c % NBUF
        if c >= NBUF:
            ocp[c - NBUF].wait()
        gcp[c] = pltpu.async_copy(
            weight_hbm.at[bidx_v.at[pl.ds(c * WCH, WCH)]], bufs[b], gsems[b])
        if c >= 1:
            pb = (c - 1) % NBUF
            gcp[c - 1].wait()
            ocp[c - 1] = pltpu.async_copy(
                bufs[pb], base_hbm.at[pl.ds(base + (c - 1) * WCH, WCH)],
                osems[pb])
    gcp[NCH - 1].wait()
    ocp[NCH - 1] = pltpu.async_copy(
        bufs[(NCH - 1) % NBUF],
        base_hbm.at[pl.ds(base + (NCH - 1) * WCH, WCH)],
        osems[(NCH - 1) % NBUF])
    for c in range(NCH - NBUF, NCH):
        ocp[c].wait()


def _sc_w_gather(bidx, weight):
    mesh = plsc.VectorSubcoreMesh(core_axis_name="c", subcore_axis_name="s")
    return pl.kernel(
        _sc_w_body,
        out_type=jax.ShapeDtypeStruct((T, EMBED_DIM), jnp.float32),
        mesh=mesh,
        scratch_types=[
            pltpu.VMEM((B_PER_W,), jnp.int32),
            pltpu.VMEM((WCH, EMBED_DIM), jnp.float32),
            pltpu.VMEM((WCH, EMBED_DIM), jnp.float32),
            pltpu.VMEM((WCH, EMBED_DIM), jnp.float32),
            pltpu.SemaphoreType.DMA,
            pltpu.SemaphoreType.DMA,
            pltpu.SemaphoreType.DMA,
            pltpu.SemaphoreType.DMA,
            pltpu.SemaphoreType.DMA,
            pltpu.SemaphoreType.DMA,
        ],
    )(bidx, weight)


def _tc_body(g_ref, bflat_ref, base_ref, out_ref):
    lora = jnp.dot(g_ref[...], bflat_ref[...],
                   preferred_element_type=jnp.float32)
    out_ref[...] = base_ref[...] + lora


def _tc_call(g, bflat, base_out):
    return pl.pallas_call(
        _tc_body,
        grid=(NBLK,),
        in_specs=[
            pl.BlockSpec((TOK_BLK, MAX_L * RANK), lambda i: (i, 0)),
            pl.BlockSpec((MAX_L * RANK, EMBED_DIM), lambda i: (0, 0)),
            pl.BlockSpec((TOK_BLK, EMBED_DIM), lambda i: (i, 0)),
        ],
        out_specs=pl.BlockSpec((TOK_BLK, EMBED_DIM), lambda i: (i, 0)),
        out_shape=jax.ShapeDtypeStruct((T, EMBED_DIM), jnp.float32),
    )(g, bflat, base_out)


def kernel(x, lora_indices, weight, lora_a_stacked_2d, lora_b_stacked):
    g, bidx = _sc_a_gather(x, lora_indices, lora_a_stacked_2d)
    base_out = _sc_w_gather(bidx, weight)
    bflat = jnp.transpose(lora_b_stacked[:, 0], (0, 2, 1)).reshape(
        MAX_L * RANK, EMBED_DIM)
    return _tc_call(g, bflat, base_out)


# speedup vs baseline: 3.7350x; 2.2761x over previous
import jax
import jax.numpy as jnp
from jax import lax
from jax.experimental import pallas as pl
from jax.experimental.pallas import tpu as pltpu
from jax.experimental.pallas import tpu_sc as plsc

ORG_VOCAB = 100000
EXTRA_VOCAB = 256
FULL_VOCAB = ORG_VOCAB + EXTRA_VOCAB
EMBED_DIM = 1024
MAX_L = 8
RANK = 16
T = 16384

NC = 2
NS = 16
NW = NC * NS
B_PER_W = T // NW
GCHUNK = 128
WCH = 32
NCH = B_PER_W // WCH
NBUF = 3

TOK_BLK = 1024
NBLK = T // TOK_BLK


def _sc_a_body(x_hbm, l_hbm, lora_a_t_hbm, g_hbm, bidx_hbm,
               x_v, l_v, idx_v, bidx_v, sb0, sb1, g_v, sem0, sem1):
    wid = lax.axis_index("s") * NC + lax.axis_index("c")
    base = wid * B_PER_W
    pltpu.sync_copy(x_hbm.at[pl.ds(base, B_PER_W)], x_v)
    pltpu.sync_copy(l_hbm.at[pl.ds(base, B_PER_W)], l_v)

    def step(i, carry):
        xs = x_v[pl.ds(i * 16, 16)]
        ls = l_v[pl.ds(i * 16, 16)]
        idx_v[pl.ds(i * 16, 16)] = xs + ls * FULL_VOCAB
        extra = jnp.where(xs > ORG_VOCAB - 1, ls * EXTRA_VOCAB, 0)
        bidx_v[pl.ds(i * 16, 16)] = xs + extra
        return carry

    lax.fori_loop(0, B_PER_W // 16, step, 0)

    zeros16 = jnp.zeros((16,), jnp.float32)
    rows16 = lax.iota(jnp.int32, 16)
    sbs = (sb0, sb1)
    sems = (sem0, sem1)

    def make_dstep(off):
        def dstep(gi, carry):
            g0 = off + gi * 16
            cvec = idx_v[pl.ds(g0, 16)]
            lvec = l_v[pl.ds(g0, 16)]
            tstart = (cvec >> 7) << 7
            cmv = cvec & 127
            for h in range(2):
                for i in range(8):
                    k = h * 8 + i
                    start = pl.multiple_of(tstart[k], 128)
                    pltpu.async_copy(
                        lora_a_t_hbm.at[:, pl.ds(start, 128)],
                        sbs[h].at[pl.ds(i * RANK, RANK), pl.ds(0, 128)],
                        sems[h],
                    )
            for h in range(2):
                for i in range(8):
                    pltpu.make_async_copy(
                        lora_a_t_hbm.at[:, pl.ds(0, 128)],
                        sbs[h].at[pl.ds(i * RANK, RANK), pl.ds(0, 128)],
                        sems[h],
                    ).wait()
                for i in range(8):
                    k = h * 8 + i
                    t = gi * 16 + k
                    cm = cmv[k]
                    lt = lvec[k]
                    arow = plsc.load_gather(
                        sbs[h],
                        [i * RANK + rows16, jnp.broadcast_to(cm, (16,))])
                    for j in range(MAX_L):
                        g_v[t, pl.ds(j * RANK, RANK)] = zeros16
                    g_v[t, pl.ds(lt * RANK, RANK)] = arow
            return carry
        return dstep

    half = B_PER_W // 2
    for p in range(2):
        lax.fori_loop(0, half // 16, make_dstep(p * half), 0)
        pltpu.sync_copy(g_v, g_hbm.at[pl.ds(base + p * half, half)])
    pltpu.sync_copy(bidx_v, bidx_hbm.at[pl.ds(base, B_PER_W)])


def _sc_a_gather(x, lora_indices, lora_a_t):
    mesh = plsc.VectorSubcoreMesh(core_axis_name="c", subcore_axis_name="s")
    return pl.kernel(
        _sc_a_body,
        out_type=[
            jax.ShapeDtypeStruct((T, MAX_L * RANK), jnp.float32),
            jax.ShapeDtypeStruct((T,), jnp.int32),
        ],
        mesh=mesh,
        scratch_types=[
            pltpu.VMEM((B_PER_W,), jnp.int32),
            pltpu.VMEM((B_PER_W,), jnp.int32),
            pltpu.VMEM((B_PER_W,), jnp.int32),
            pltpu.VMEM((B_PER_W,), jnp.int32),
            pltpu.VMEM((8 * RANK, 129), jnp.float32),
            pltpu.VMEM((8 * RANK, 129), jnp.float32),
            pltpu.VMEM((B_PER_W // 2, MAX_L * RANK), jnp.float32),
            pltpu.SemaphoreType.DMA,
            pltpu.SemaphoreType.DMA,
        ],
        compiler_params=pltpu.CompilerParams(needs_layout_passes=False),
    )(x, lora_indices, lora_a_t)


def _sc_w_body(bidx_hbm, weight_hbm, base_hbm,
               bidx_v, buf0, buf1, buf2, gs0, gs1, gs2, os0, os1, os2):
    wid = lax.axis_index("s") * NC + lax.axis_index("c")
    base = wid * B_PER_W
    pltpu.sync_copy(bidx_hbm.at[pl.ds(base, B_PER_W)], bidx_v)

    bufs = [buf0, buf1, buf2]
    gsems = [gs0, gs1, gs2]
    osems = [os0, os1, os2]
    gcp = {}
    ocp = {}
    for c in range(NCH):
        b = c % NBUF
        if c >= NBUF:
            ocp[c - NBUF].wait()
        gcp[c] = pltpu.async_copy(
            weight_hbm.at[bidx_v.at[pl.ds(c * WCH, WCH)]], bufs[b], gsems[b])
        if c >= 1:
            pb = (c - 1) % NBUF
            gcp[c - 1].wait()
            ocp[c - 1] = pltpu.async_copy(
                bufs[pb], base_hbm.at[pl.ds(base + (c - 1) * WCH, WCH)],
                osems[pb])
    gcp[NCH - 1].wait()
    ocp[NCH - 1] = pltpu.async_copy(
        bufs[(NCH - 1) % NBUF],
        base_hbm.at[pl.ds(base + (NCH - 1) * WCH, WCH)],
        osems[(NCH - 1) % NBUF])
    for c in range(NCH - NBUF, NCH):
        ocp[c].wait()


def _sc_w_gather(bidx, weight):
    mesh = plsc.VectorSubcoreMesh(core_axis_name="c", subcore_axis_name="s")
    return pl.kernel(
        _sc_w_body,
        out_type=jax.ShapeDtypeStruct((T, EMBED_DIM), jnp.float32),
        mesh=mesh,
        scratch_types=[
            pltpu.VMEM((B_PER_W,), jnp.int32),
            pltpu.VMEM((WCH, EMBED_DIM), jnp.float32),
            pltpu.VMEM((WCH, EMBED_DIM), jnp.float32),
            pltpu.VMEM((WCH, EMBED_DIM), jnp.float32),
            pltpu.SemaphoreType.DMA,
            pltpu.SemaphoreType.DMA,
            pltpu.SemaphoreType.DMA,
            pltpu.SemaphoreType.DMA,
            pltpu.SemaphoreType.DMA,
            pltpu.SemaphoreType.DMA,
        ],
    )(bidx, weight)


def _tc_body(g_ref, bflat_ref, base_ref, out_ref):
    lora = jnp.dot(g_ref[...], bflat_ref[...],
                   preferred_element_type=jnp.float32)
    out_ref[...] = base_ref[...] + lora


def _tc_call(g, bflat, base_out):
    return pl.pallas_call(
        _tc_body,
        grid=(NBLK,),
        in_specs=[
            pl.BlockSpec((TOK_BLK, MAX_L * RANK), lambda i: (i, 0)),
            pl.BlockSpec((MAX_L * RANK, EMBED_DIM), lambda i: (0, 0)),
            pl.BlockSpec((TOK_BLK, EMBED_DIM), lambda i: (i, 0)),
        ],
        out_specs=pl.BlockSpec((TOK_BLK, EMBED_DIM), lambda i: (i, 0)),
        out_shape=jax.ShapeDtypeStruct((T, EMBED_DIM), jnp.float32),
    )(g, bflat, base_out)


def kernel(x, lora_indices, weight, lora_a_stacked_2d, lora_b_stacked):
    g, bidx = _sc_a_gather(x, lora_indices, lora_a_stacked_2d.T)
    base_out = _sc_w_gather(bidx, weight)
    bflat = jnp.transpose(lora_b_stacked[:, 0], (0, 2, 1)).reshape(
        MAX_L * RANK, EMBED_DIM)
    return _tc_call(g, bflat, base_out)
